# initial kernel scaffold (unmeasured)
import jax
import jax.numpy as jnp
from jax import lax
from jax.experimental import pallas as pl
from jax.experimental.pallas import tpu as pltpu

N_DEV = 16


def kernel(x, w_mat, scale_x, scale_w):
    m_global, k_local = x.shape
    k_global, n = w_mat.shape
    m_per = m_global // N_DEV

    def body(x_ref, w_ref, sx_ref, sw_ref, out_ref,
             xblk_ref, acc_ref, send_sems, recv_sems):
        my = lax.axis_index("i")

        sends = []
        for j in range(1, N_DEV):
            tgt = lax.rem(my + j, N_DEV)
            rdma = pltpu.make_async_remote_copy(
                src_ref=x_ref.at[pl.ds(tgt * m_per, m_per), :],
                dst_ref=xblk_ref.at[my],
                send_sem=send_sems.at[j],
                recv_sem=recv_sems.at[my],
                device_id=(tgt,),
                device_id_type=pl.DeviceIdType.MESH,
            )
            rdma.start()
            sends.append(rdma)

        acc_ref[...] = jnp.dot(
            x_ref[pl.ds(my * m_per, m_per), :],
            w_ref[pl.ds(my * k_local, k_local), :],
            preferred_element_type=jnp.int32,
        )

        for j in range(1, N_DEV):
            src = lax.rem(my - j + N_DEV, N_DEV)
            recv = pltpu.make_async_remote_copy(
                src_ref=x_ref.at[pl.ds(0, m_per), :],
                dst_ref=xblk_ref.at[src],
                send_sem=send_sems.at[0],
                recv_sem=recv_sems.at[src],
                device_id=(src,),
                device_id_type=pl.DeviceIdType.MESH,
            )
            recv.wait_recv()
            acc_ref[...] += jnp.dot(
                xblk_ref[src],
                w_ref[pl.ds(src * k_local, k_local), :],
                preferred_element_type=jnp.int32,
            )

        scale = sx_ref[0] * sw_ref[0]
        out_ref[...] = acc_ref[...].astype(jnp.float32) * scale

        for rdma in sends:
            rdma.wait_send()

    return pl.pallas_call(
        body,
        out_shape=jax.ShapeDtypeStruct((m_per, n), jnp.float32),
        in_specs=[
            pl.BlockSpec(memory_space=pltpu.VMEM),
            pl.BlockSpec(memory_space=pltpu.VMEM),
            pl.BlockSpec(memory_space=pltpu.SMEM),
            pl.BlockSpec(memory_space=pltpu.SMEM),
        ],
        out_specs=pl.BlockSpec(memory_space=pltpu.VMEM),
        scratch_shapes=[
            pltpu.VMEM((N_DEV, m_per, k_local), jnp.int8),
            pltpu.VMEM((m_per, n), jnp.int32),
            pltpu.SemaphoreType.DMA((N_DEV,)),
            pltpu.SemaphoreType.DMA((N_DEV,)),
        ],
        compiler_params=pltpu.CompilerParams(collective_id=0),
    )(x, w_mat, scale_x, scale_w)


# baseline (device time: 29941 ns/iter reference)
import jax
import jax.numpy as jnp
from jax import lax
from jax.experimental import pallas as pl
from jax.experimental.pallas import tpu as pltpu

N_DEV = 16


def kernel(x, w_mat, scale_x, scale_w):
    m_global, k_local = x.shape
    k_global, n = w_mat.shape
    m_per = m_global // N_DEV

    def body(x_ref, w_ref, sx_ref, sw_ref, out_ref,
             xblk_ref, acc_ref, send_sems, recv_sems):
        my = lax.axis_index("i")

        sends = []
        for j in range(1, N_DEV):
            tgt = lax.rem(my + j, N_DEV)
            rdma = pltpu.make_async_remote_copy(
                src_ref=x_ref.at[pl.ds(tgt * m_per, m_per), :],
                dst_ref=xblk_ref.at[my],
                send_sem=send_sems.at[j],
                recv_sem=recv_sems.at[my],
                device_id=(tgt,),
                device_id_type=pl.DeviceIdType.MESH,
            )
            rdma.start()
            sends.append(rdma)

        acc_ref[...] = jnp.dot(
            x_ref[pl.ds(my * m_per, m_per), :],
            w_ref[pl.ds(my * k_local, k_local), :],
            preferred_element_type=jnp.int32,
        )

        for j in range(1, N_DEV):
            src = lax.rem(my - j + N_DEV, N_DEV)
            recv = pltpu.make_async_remote_copy(
                src_ref=x_ref.at[pl.ds(0, m_per), :],
                dst_ref=xblk_ref.at[src],
                send_sem=send_sems.at[0],
                recv_sem=recv_sems.at[src],
                device_id=(src,),
                device_id_type=pl.DeviceIdType.MESH,
            )
            recv.wait_recv()
            acc_ref[...] += jnp.dot(
                xblk_ref[src],
                w_ref[pl.ds(src * k_local, k_local), :],
                preferred_element_type=jnp.int32,
            )

        scale = sx_ref[0] * sw_ref[0]
        out_ref[...] = acc_ref[...].astype(jnp.float32) * scale

        for rdma in sends:
            rdma.wait_send()

    return pl.pallas_call(
        body,
        out_shape=jax.ShapeDtypeStruct((m_per, n), jnp.float32),
        in_specs=[
            pl.BlockSpec(memory_space=pltpu.VMEM),
            pl.BlockSpec(memory_space=pltpu.VMEM),
            pl.BlockSpec(memory_space=pltpu.SMEM),
            pl.BlockSpec(memory_space=pltpu.SMEM),
        ],
        out_specs=pl.BlockSpec(memory_space=pltpu.VMEM),
        scratch_shapes=[
            pltpu.VMEM((N_DEV, m_per, k_local), jnp.int8),
            pltpu.VMEM((m_per, n), jnp.int32),
            pltpu.SemaphoreType.DMA((N_DEV,)),
            pltpu.SemaphoreType.DMA((N_DEV,)),
        ],
    )(x, w_mat, scale_x, scale_w)
